# Initial kernel scaffold; baseline (speedup 1.0000x reference)
import jax
import jax.numpy as jnp
from jax.experimental import pallas as pl


def kernel(x, cb_index, codebooks, linear_w, linear_b, bn0_g, bn0_b, w1, b1, g1, be1, w2, b2, g2, be2, w3, b3):
    raise NotImplementedError("write your pallas kernel here")


# speedup vs baseline: 3.1201x; 3.1201x over previous
import functools

import jax
import jax.numpy as jnp
import numpy as np
from jax import lax
from jax.experimental import pallas as pl
from jax.experimental.pallas import tpu as pltpu
from jax.experimental.pallas import tpu_sc as plsc

F = 26
DIM = 128
M = 8
K = 256
B = 4096
PLEN = DIM // M
TOTAL = 100000 * F
EPS = 1e-5

NSC = 2
NSUB = 16
NW = NSC * NSUB
QROWS = B // 4
CHUNK = 64
NCHUNK = QROWS // CHUNK
CROWS = CHUNK * F // 128
LROWS = B // NW

_mesh = plsc.VectorSubcoreMesh(core_axis_name="c", subcore_axis_name="s")


def _sc_body(gidx_h, gidxT_h, cb_h, tbl_h, linw_h, cross_h, lin_h,
             tblv, idxv, rowv, outv, lidxv, linrowv, loutv, sem):
    ci = lax.axis_index("c")
    si = lax.axis_index("s")
    wid = si * NSC + ci
    j = lax.rem(wid, M)
    q = wid // M
    j16 = j * PLEN

    pltpu.sync_copy(tbl_h.at[j], tblv)

    iota = lax.iota(jnp.int32, 16)
    zeros_i = jnp.zeros((16,), jnp.int32)
    jv = zeros_i + j

    @pl.loop(0, NCHUNK)
    def _fm(chunk):
        b0 = q * QROWS + chunk * CHUNK
        r0 = q * (QROWS * F // 128) + chunk * CROWS
        pltpu.sync_copy(gidx_h.at[pl.ds(r0, CROWS)], idxv)
        cps = [
            pltpu.async_copy(cb_h.at[idxv.at[i]],
                             rowv.at[pl.ds(i * 128, 128)], sem)
            for i in range(CROWS)
        ]
        for cp in cps:
            cp.wait()

        @pl.loop(0, CHUNK // 16)
        def _grp(g):
            rowbase = (g * 16 + iota) * F
            acc_s = [jnp.zeros((16,), jnp.float32) for _ in range(PLEN)]
            acc_t = [jnp.zeros((16,), jnp.float32) for _ in range(PLEN)]
            for f in range(F):
                codes = plsc.load_gather(rowv, [rowbase + f, jv])
                base = codes * PLEN + (f * K * PLEN)
                for d in range(PLEN):
                    v = plsc.load_gather(tblv, [base + d])
                    acc_s[d] = acc_s[d] + v
                    acc_t[d] = acc_t[d] + v * v
            orow = g * 16 + iota
            for d in range(PLEN):
                cv = 0.5 * (acc_s[d] * acc_s[d] - acc_t[d])
                plsc.store_scatter(outv, [orow, zeros_i + d], cv)

        pltpu.sync_copy(outv, cross_h.at[pl.ds(b0, CHUNK), pl.ds(j16, PLEN)])

    pltpu.sync_copy(gidxT_h.at[wid], lidxv)
    lcps = [
        pltpu.async_copy(linw_h.at[lidxv.at[f]],
                         linrowv.at[pl.ds(f * 128, 128)], sem)
        for f in range(F)
    ]
    for cp in lcps:
        cp.wait()
    for g in range(LROWS // 16):
        acc0 = jnp.zeros((16,), jnp.float32)
        acc1 = jnp.zeros((16,), jnp.float32)
        for f in range(F):
            v = linrowv[pl.ds(f * 128 + g * 16, 16)]
            if f % 2 == 0:
                acc0 = acc0 + v
            else:
                acc1 = acc1 + v
        loutv[pl.ds(g * 16, 16)] = acc0 + acc1
    pltpu.sync_copy(loutv, lin_h.at[pl.ds(wid * LROWS, LROWS)])


@jax.jit
def _sc_call(gidx, gidxT, cb_index, tbl, linw):
    k = functools.partial(
        pl.kernel,
        out_type=[jax.ShapeDtypeStruct((B, DIM), jnp.float32),
                  jax.ShapeDtypeStruct((B,), jnp.float32)],
        mesh=_mesh,
        scratch_types=[
            pltpu.VMEM((F * K * PLEN,), jnp.float32),
            pltpu.VMEM((CROWS, 128), jnp.int32),
            pltpu.VMEM((CHUNK * F, M), jnp.int32),
            pltpu.VMEM((CHUNK, PLEN), jnp.float32),
            pltpu.VMEM((F, 128), jnp.int32),
            pltpu.VMEM((F * 128,), jnp.float32),
            pltpu.VMEM((LROWS,), jnp.float32),
            pltpu.SemaphoreType.DMA,
        ],
        compiler_params=pltpu.CompilerParams(use_tc_tiling_on_sc=False,
                                             needs_layout_passes=False),
    )(_sc_body)
    return k(gidx, gidxT, cb_index, tbl, linw)


def _mlp_body(cross_ref, lin_ref, w1_ref, b1_ref, w2_ref, b2_ref,
              w3_ref, b3_ref, out_ref):
    h = jnp.dot(cross_ref[...], w1_ref[...],
                preferred_element_type=jnp.float32) + b1_ref[...]
    h = jnp.maximum(h, 0.0)
    h = jnp.dot(h, w2_ref[...],
                preferred_element_type=jnp.float32) + b2_ref[...]
    h = jnp.maximum(h, 0.0)
    o = jnp.dot(h, w3_ref[...], preferred_element_type=jnp.float32)
    out_ref[...] = o + b3_ref[...] + lin_ref[...]


def _mlp_call(cross, lin2, w1f, beta1, w2f, beta2, w3f, beta3):
    bb = 512
    return pl.pallas_call(
        _mlp_body,
        grid=(B // bb,),
        in_specs=[
            pl.BlockSpec((bb, DIM), lambda i: (i, 0)),
            pl.BlockSpec((bb, 1), lambda i: (i, 0)),
            pl.BlockSpec((DIM, 1024), lambda i: (0, 0)),
            pl.BlockSpec((1, 1024), lambda i: (0, 0)),
            pl.BlockSpec((1024, 512), lambda i: (0, 0)),
            pl.BlockSpec((1, 512), lambda i: (0, 0)),
            pl.BlockSpec((512, 1), lambda i: (0, 0)),
            pl.BlockSpec((1, 1), lambda i: (0, 0)),
        ],
        out_specs=pl.BlockSpec((bb, 1), lambda i: (i, 0)),
        out_shape=jax.ShapeDtypeStruct((B, 1), jnp.float32),
    )(cross, lin2, w1f, beta1, w2f, beta2, w3f, beta3)


_OFFS = np.concatenate([[0], np.cumsum([100000] * F)[:-1]]).astype(np.int32)


def kernel(x, cb_index, codebooks, linear_w, linear_b, bn0_g, bn0_b,
           w1, b1, g1, be1, w2, b2, g2, be2, w3, b3):
    gflat = (x + jnp.asarray(_OFFS)[None, :]).astype(jnp.int32)
    gidx = gflat.reshape(B * F // 128, 128)
    gidxT = gflat.T.reshape(F, NW, LROWS).transpose(1, 0, 2)
    tbl = (codebooks.reshape(F * K, M, PLEN)
           .transpose(1, 0, 2).reshape(M, F * K * PLEN))
    linw = linear_w.reshape(-1)

    cross, lin = _sc_call(gidx, gidxT, cb_index, tbl, linw)

    c = 1.0 / jnp.sqrt(jnp.float32(1.0 + EPS))
    w1f = (w1 * (c * bn0_g)[None, :]).T * (c * g1)[None, :]
    beta1 = (w1 @ bn0_b + b1) * (c * g1) + be1
    w2f = w2.T * (c * g2)[None, :]
    beta2 = b2 * (c * g2) + be2
    w3f = w3.T
    beta3 = (b3 + linear_b).reshape(1, 1)

    out = _mlp_call(cross, lin.reshape(B, 1), w1f, beta1.reshape(1, -1),
                    w2f, beta2.reshape(1, -1), w3f, beta3)
    return out.reshape(B)
